# baseline (speedup 1.0000x reference)
import functools

import jax
import jax.numpy as jnp
from jax import lax
from jax.experimental import pallas as pl
from jax.experimental.pallas import tpu as pltpu
from jax.experimental.pallas import tpu_sc as plsc

K = 10


def _tc_head_body(a_ref, w_ref, b_ref, tags_ref, topi_ref):
    logits = jnp.dot(a_ref[...], w_ref[...], preferred_element_type=jnp.float32)
    logits = logits + b_ref[...]
    m = jnp.max(logits, axis=-1, keepdims=True)
    e = jnp.exp(logits - m)
    tags_ref[...] = e / jnp.sum(e, axis=-1, keepdims=True)
    c = logits.shape[-1]
    iota = lax.broadcasted_iota(jnp.int32, logits.shape, 1)
    cur = logits
    for j in range(K):
        mx = jnp.max(cur, axis=-1, keepdims=True)
        am = jnp.min(jnp.where(cur == mx, iota, c), axis=-1, keepdims=True)
        topi_ref[:, pl.ds(j, 1)] = am
        cur = jnp.where(iota == am, -jnp.inf, cur)


def _tc_head(feats, w, b):
    bsz, d = feats.shape
    c = w.shape[1]
    bm = 256
    return pl.pallas_call(
        _tc_head_body,
        grid=(bsz // bm,),
        in_specs=[
            pl.BlockSpec((bm, d), lambda i: (i, 0)),
            pl.BlockSpec((d, c), lambda i: (0, 0)),
            pl.BlockSpec((1, c), lambda i: (0, 0)),
        ],
        out_specs=[
            pl.BlockSpec((bm, c), lambda i: (i, 0)),
            pl.BlockSpec((bm, K), lambda i: (i, 0)),
        ],
        out_shape=[
            jax.ShapeDtypeStruct((bsz, c), jnp.float32),
            jax.ShapeDtypeStruct((bsz, K), jnp.int32),
        ],
    )(feats, w, b.reshape(1, c))


def _sc_gather(embed, idx_flat):
    n = idx_flat.shape[0]
    d = embed.shape[1]
    nrow = embed.shape[0]
    bsz = n // K
    info = plsc.get_sparse_core_info()
    nc, ns = info.num_cores, info.num_subcores
    nw = nc * ns
    b_per_w = bsz // nw
    n_per_w = b_per_w * K
    cb = 2
    cf = cb * K
    nbuf = 2
    n_chunks = b_per_w // cb
    lanes = info.num_lanes

    mesh = plsc.VectorSubcoreMesh(core_axis_name="c", subcore_axis_name="s")

    @functools.partial(
        pl.kernel,
        mesh=mesh,
        out_type=jax.ShapeDtypeStruct((bsz, K, d), jnp.float32),
        scratch_types=[
            pltpu.VMEM((nrow, d), jnp.float32),
            pltpu.VMEM((n_per_w + 16, ), jnp.int32),
            pltpu.VMEM((nbuf, cb, K, d), jnp.float32),
            pltpu.SemaphoreType.DMA,
            pltpu.SemaphoreType.DMA,
        ],
    )
    def gather_kernel(embed_hbm, idx_hbm, out_hbm, table_v, idx_v, stage_v,
                      wsem0, wsem1):
        sid = lax.axis_index("s")
        wid = sid * nc + lax.axis_index("c")
        wb0 = wid * b_per_w

        pltpu.sync_copy(embed_hbm, table_v)
        pltpu.sync_copy(idx_hbm.at[pl.ds(wb0 * K, n_per_w)],
                        idx_v.at[pl.ds(0, n_per_w)])
        wsems = (wsem0, wsem1)

        def chunk_body(ci, carry):
            f0 = pl.multiple_of(ci * (nbuf * cf), cf)
            for s in range(nbuf):
                c = nbuf * ci + s
                base = f0 + s * cf

                @pl.when(ci > 0)
                def _():
                    pltpu.make_async_copy(
                        stage_v.at[s], out_hbm.at[pl.ds(0, cb)], wsems[s]
                    ).wait()

                grp = None
                for r in range(cf):
                    if r % lanes == 0:
                        grp = idx_v[pl.ds(base + r, lanes)]
                    row = grp[r % lanes]
                    for j in range(d // lanes):
                        stage_v[s, r // K, r % K, pl.ds(j * lanes, lanes)] = (
                            table_v[row, pl.ds(j * lanes, lanes)]
                        )
                pltpu.async_copy(
                    stage_v.at[s],
                    out_hbm.at[pl.ds(wb0 + c * cb, cb)],
                    wsems[s],
                )
            return carry

        lax.fori_loop(0, n_chunks // nbuf, chunk_body, 0)
        for s in range(nbuf):
            pltpu.make_async_copy(
                stage_v.at[s], out_hbm.at[pl.ds(0, cb)], wsems[s]
            ).wait()

    return gather_kernel(embed, idx_flat)


def kernel(avg_features, W, b, embed):
    tags, topi = _tc_head(avg_features, W, b)
    rows = _sc_gather(embed, topi.reshape(-1))
    return tags, rows


# speedup vs baseline: 1.0153x; 1.0153x over previous
import functools

import jax
import jax.numpy as jnp
from jax import lax
from jax.experimental import pallas as pl
from jax.experimental.pallas import tpu as pltpu
from jax.experimental.pallas import tpu_sc as plsc

K = 10


def _tc_head_body(a_ref, w_ref, b_ref, tags_ref, topi_ref):
    logits = jnp.dot(a_ref[...], w_ref[...], preferred_element_type=jnp.float32)
    logits = logits + b_ref[...]
    m = jnp.max(logits, axis=-1, keepdims=True)
    e = jnp.exp(logits - m)
    tags_ref[...] = e / jnp.sum(e, axis=-1, keepdims=True)
    c = logits.shape[-1]
    iota = lax.broadcasted_iota(jnp.int32, logits.shape, 1)
    cur = logits
    for j in range(K):
        mx = jnp.max(cur, axis=-1, keepdims=True)
        am = jnp.min(jnp.where(cur == mx, iota, c), axis=-1, keepdims=True)
        topi_ref[:, pl.ds(j, 1)] = am
        cur = jnp.where(iota == am, -jnp.inf, cur)


def _tc_head(feats, w, b):
    bsz, d = feats.shape
    c = w.shape[1]
    bm = 256
    return pl.pallas_call(
        _tc_head_body,
        grid=(bsz // bm,),
        in_specs=[
            pl.BlockSpec((bm, d), lambda i: (i, 0)),
            pl.BlockSpec((d, c), lambda i: (0, 0)),
            pl.BlockSpec((1, c), lambda i: (0, 0)),
        ],
        out_specs=[
            pl.BlockSpec((bm, c), lambda i: (i, 0)),
            pl.BlockSpec((bm, K), lambda i: (i, 0)),
        ],
        out_shape=[
            jax.ShapeDtypeStruct((bsz, c), jnp.float32),
            jax.ShapeDtypeStruct((bsz, K), jnp.int32),
        ],
    )(feats, w, b.reshape(1, c))


def _sc_gather(embed, idx_t, out_bsz, out_b0):
    bsz = idx_t.shape[1]
    d = embed.shape[1]
    info = plsc.get_sparse_core_info()
    nc, ns = info.num_cores, info.num_subcores
    nw = nc * ns
    b_per_w = bsz // nw
    cb = 64
    nbuf = 2
    n_chunks = b_per_w // cb

    mesh = plsc.VectorSubcoreMesh(core_axis_name="c", subcore_axis_name="s")

    @functools.partial(
        pl.kernel,
        mesh=mesh,
        out_type=jax.ShapeDtypeStruct((out_bsz, K, d), jnp.float32),
        scratch_types=[
            pltpu.VMEM((K, b_per_w), jnp.int32),
            pltpu.VMEM((nbuf, cb, d), jnp.float32),
            pltpu.SemaphoreType.DMA,
            pltpu.SemaphoreType.DMA,
            pltpu.SemaphoreType.DMA,
        ],
    )
    def gather_kernel(embed_hbm, idx_hbm, out_hbm, idx_v, rows_v,
                      gsem, wsem0, wsem1):
        sid = lax.axis_index("s")
        wid = sid * nc + lax.axis_index("c")
        wb0 = wid * b_per_w

        pltpu.sync_copy(idx_hbm.at[:, pl.ds(wb0, b_per_w)], idx_v)
        wsems = (wsem0, wsem1)

        def chunk_body(ci, carry):
            c0 = pl.multiple_of(ci * cb, cb)
            for k in range(K):
                s = k % nbuf

                @pl.when((ci > 0) | (k >= nbuf))
                def _():
                    pltpu.make_async_copy(
                        rows_v.at[s], out_hbm.at[pl.ds(0, cb), 0], wsems[s]
                    ).wait()

                pltpu.async_copy(
                    embed_hbm.at[idx_v.at[k, pl.ds(c0, cb)]],
                    rows_v.at[s], gsem,
                ).wait()
                pltpu.async_copy(
                    rows_v.at[s],
                    out_hbm.at[pl.ds(out_b0 + wb0 + c0, cb), k],
                    wsems[s],
                )
            return carry

        lax.fori_loop(0, n_chunks, chunk_body, 0)
        for s in range(nbuf):
            pltpu.make_async_copy(
                rows_v.at[s], out_hbm.at[pl.ds(0, cb), 0], wsems[s]
            ).wait()

    return gather_kernel(embed, idx_t)


def kernel(avg_features, W, b, embed):
    bsz = avg_features.shape[0]
    h = bsz // 2
    tags1, topi1 = _tc_head(avg_features[:h], W, b)
    rows1 = _sc_gather(embed, topi1.T, h, 0)
    tags2, topi2 = _tc_head(avg_features[h:], W, b)
    rows2 = _sc_gather(embed, topi2.T, h, 0)
    tags = jnp.concatenate([tags1, tags2], axis=0)
    rows = jnp.concatenate([rows1, rows2], axis=0)
    return tags, rows


# speedup vs baseline: 1.2954x; 1.2758x over previous
import functools

import jax
import jax.numpy as jnp
from jax import lax
from jax.experimental import pallas as pl
from jax.experimental.pallas import tpu as pltpu
from jax.experimental.pallas import tpu_sc as plsc

K = 10


def _tc_head_body(a_ref, w_ref, b_ref, tags_ref, topi_ref):
    logits = jnp.dot(a_ref[...], w_ref[...], preferred_element_type=jnp.float32)
    logits = logits + b_ref[...]
    m = jnp.max(logits, axis=-1, keepdims=True)
    e = jnp.exp(logits - m)
    tags_ref[...] = e / jnp.sum(e, axis=-1, keepdims=True)
    c = logits.shape[-1]
    iota = lax.broadcasted_iota(jnp.int32, logits.shape, 1)
    cur = logits
    for j in range(K):
        mx = jnp.max(cur, axis=-1, keepdims=True)
        am = jnp.min(jnp.where(cur == mx, iota, c), axis=-1, keepdims=True)
        topi_ref[:, pl.ds(j, 1)] = am
        cur = jnp.where(iota == am, -jnp.inf, cur)


def _tc_head(feats, w, b):
    bsz, d = feats.shape
    c = w.shape[1]
    bm = 256
    return pl.pallas_call(
        _tc_head_body,
        grid=(bsz // bm,),
        in_specs=[
            pl.BlockSpec((bm, d), lambda i: (i, 0)),
            pl.BlockSpec((d, c), lambda i: (0, 0)),
            pl.BlockSpec((1, c), lambda i: (0, 0)),
        ],
        out_specs=[
            pl.BlockSpec((bm, c), lambda i: (i, 0)),
            pl.BlockSpec((bm, K), lambda i: (i, 0)),
        ],
        out_shape=[
            jax.ShapeDtypeStruct((bsz, c), jnp.float32),
            jax.ShapeDtypeStruct((bsz, K), jnp.int32),
        ],
    )(feats, w, b.reshape(1, c))


def _sc_gather(embed, idx_t, out_bsz, out_b0):
    bsz = idx_t.shape[1]
    d = embed.shape[1]
    info = plsc.get_sparse_core_info()
    nc, ns = info.num_cores, info.num_subcores
    nw = nc * ns
    b_per_w = bsz // nw
    cb = 64
    nbuf = 2
    n_chunks = b_per_w // cb

    mesh = plsc.VectorSubcoreMesh(core_axis_name="c", subcore_axis_name="s")

    @functools.partial(
        pl.kernel,
        mesh=mesh,
        out_type=jax.ShapeDtypeStruct((out_bsz, K, d), jnp.float32),
        scratch_types=[
            pltpu.VMEM((K, b_per_w), jnp.int32),
            pltpu.VMEM((nbuf, cb, d), jnp.float32),
            pltpu.SemaphoreType.DMA,
            pltpu.SemaphoreType.DMA,
            pltpu.SemaphoreType.DMA,
            pltpu.SemaphoreType.DMA,
        ],
    )
    def gather_kernel(embed_hbm, idx_hbm, out_hbm, idx_v, rows_v,
                      gsem0, gsem1, wsem0, wsem1):
        sid = lax.axis_index("s")
        wid = sid * nc + lax.axis_index("c")
        wb0 = wid * b_per_w

        pltpu.sync_copy(idx_hbm.at[:, pl.ds(wb0, b_per_w)], idx_v)
        wsems = (wsem0, wsem1)
        gsems = (gsem0, gsem1)

        def chunk_body(ci, carry):
            c0 = pl.multiple_of(ci * cb, cb)

            def gsrc(k):
                return embed_hbm.at[idx_v.at[k, pl.ds(c0, cb)]]

            def flush(k):
                s = k % nbuf
                pltpu.make_async_copy(gsrc(k), rows_v.at[s], gsems[s]).wait()
                pltpu.async_copy(
                    rows_v.at[s],
                    out_hbm.at[pl.ds(out_b0 + wb0 + c0, cb), k],
                    wsems[s],
                )

            for k in range(K):
                s = k % nbuf

                @pl.when((ci > 0) | (k >= nbuf))
                def _():
                    pltpu.make_async_copy(
                        rows_v.at[s], out_hbm.at[pl.ds(0, cb), 0], wsems[s]
                    ).wait()

                pltpu.async_copy(gsrc(k), rows_v.at[s], gsems[s])
                if k > 0:
                    flush(k - 1)
            flush(K - 1)
            return carry

        lax.fori_loop(0, n_chunks, chunk_body, 0)
        for s in range(nbuf):
            pltpu.make_async_copy(
                rows_v.at[s], out_hbm.at[pl.ds(0, cb), 0], wsems[s]
            ).wait()

    return gather_kernel(embed, idx_t)


def kernel(avg_features, W, b, embed):
    tags, topi = _tc_head(avg_features, W, b)
    rows = _sc_gather(embed, topi.T, avg_features.shape[0], 0)
    return tags, rows
